# Initial kernel scaffold; baseline (speedup 1.0000x reference)
import jax
import jax.numpy as jnp
from jax.experimental import pallas as pl


def kernel(x, edge_index, batch, W_msg, b_msg, W_out, b_out):
    raise NotImplementedError("write your pallas kernel here")


# speedup vs baseline: 6.4580x; 6.4580x over previous
import functools

import jax
import jax.numpy as jnp
from jax import lax
from jax.experimental import pallas as pl
from jax.experimental.pallas import tpu as pltpu
from jax.experimental.pallas import tpu_sc as plsc

N = 10000
E = 320000
D = 128
G = 64

NC = 2
NS = 16
NW = NC * NS
EPW = E // NW
CHUNK = 80
NCHUNK = EPW // CHUNK
NP = 10240
RPT = NP // NS


def _sc_scatter(x, src, dst, zrows, zdeg):
    mesh = plsc.VectorSubcoreMesh(
        core_axis_name="c", subcore_axis_name="s", num_cores=NC, num_subcores=NS
    )

    @functools.partial(
        pl.kernel,
        out_type=[
            jax.ShapeDtypeStruct((NC, NP, D), jnp.float32),
            jax.ShapeDtypeStruct((NW, N), jnp.float32),
        ],
        mesh=mesh,
        compiler_params=pltpu.CompilerParams(needs_layout_passes=False),
        scratch_types=[
            pltpu.VMEM((CHUNK,), jnp.int32),
            pltpu.VMEM((CHUNK,), jnp.int32),
            pltpu.VMEM((CHUNK, D), jnp.float32),
            pltpu.VMEM((N,), jnp.float32),
            pltpu.VMEM_SHARED((NP, D), jnp.float32),
            pltpu.SemaphoreType.DMA,
        ],
    )
    def k(x_hbm, src_hbm, dst_hbm, zr_hbm, zd_hbm, agg_out, deg_out,
          sidx, didx, rows, degh, aggsh, sem):
        c = lax.axis_index("c")
        s = lax.axis_index("s")
        wid = s * NC + c

        pltpu.sync_copy(zr_hbm, aggsh.at[pl.ds(s * RPT, RPT)])
        pltpu.sync_copy(zd_hbm, degh)
        plsc.subcore_barrier()

        base = wid * EPW
        ones16 = jnp.ones((16,), jnp.float32)

        def chunk(j, carry):
            off = base + j * CHUNK
            pltpu.sync_copy(src_hbm.at[pl.ds(off, CHUNK)], sidx)
            pltpu.sync_copy(dst_hbm.at[pl.ds(off, CHUNK)], didx)
            pltpu.async_copy(x_hbm.at[sidx], rows, sem).wait()
            for t in range(CHUNK // 16):
                dv = didx[pl.ds(t * 16, 16)]
                plsc.addupdate_scatter(degh, [dv], ones16)
            pltpu.sync_copy(rows, aggsh.at[didx], add=True)
            return carry

        lax.fori_loop(0, NCHUNK, chunk, 0)
        plsc.subcore_barrier()

        pltpu.sync_copy(aggsh.at[pl.ds(s * RPT, RPT)],
                        agg_out.at[c, pl.ds(s * RPT, RPT)])
        pltpu.sync_copy(degh, deg_out.at[wid])

    return k(x, src, dst, zrows, zdeg)


_ROWS = 1000
_GRID = N // _ROWS


def _dense_body(agg_ref, deg_ref, b_ref, wm_ref, bm_ref, wo_ref, bo_ref,
                out_ref, accz, accc):
    i = pl.program_id(0)

    @pl.when(i == 0)
    def _init():
        accz[...] = jnp.zeros_like(accz)
        accc[...] = jnp.zeros_like(accc)

    agg = agg_ref[0] + agg_ref[1]
    deg = jnp.sum(deg_ref[...], axis=1, keepdims=True)
    agg = agg / jnp.maximum(deg, 1.0)
    h = jnp.dot(agg, wm_ref[...], preferred_element_type=jnp.float32)
    h = jnp.maximum(h + bm_ref[...], 0.0)
    z = jnp.dot(h, wo_ref[...], preferred_element_type=jnp.float32)
    onehot = (b_ref[...] == lax.broadcasted_iota(jnp.int32, (1, G), 1))
    onehot = onehot.astype(jnp.float32)
    accz[...] += jnp.sum(onehot * z, axis=0, keepdims=True)
    accc[...] += jnp.sum(onehot, axis=0, keepdims=True)

    @pl.when(i == _GRID - 1)
    def _fin():
        out_ref[...] = accz[...] / jnp.maximum(accc[...], 1.0) + bo_ref[...]


def _dense(agg_parts, deg_t, batch2, W_msg, b_msg2, W_out, b_out2):
    return pl.pallas_call(
        _dense_body,
        grid=(_GRID,),
        in_specs=[
            pl.BlockSpec((NC, _ROWS, D), lambda i: (0, i, 0)),
            pl.BlockSpec((_ROWS, NW), lambda i: (i, 0)),
            pl.BlockSpec((_ROWS, 1), lambda i: (i, 0)),
            pl.BlockSpec((D, D), lambda i: (0, 0)),
            pl.BlockSpec((1, D), lambda i: (0, 0)),
            pl.BlockSpec((D, 1), lambda i: (0, 0)),
            pl.BlockSpec((1, 1), lambda i: (0, 0)),
        ],
        out_specs=pl.BlockSpec((1, G), lambda i: (0, 0)),
        out_shape=jax.ShapeDtypeStruct((1, G), jnp.float32),
        scratch_shapes=[
            pltpu.VMEM((1, G), jnp.float32),
            pltpu.VMEM((1, G), jnp.float32),
        ],
    )(agg_parts, deg_t, batch2, W_msg, b_msg2, W_out, b_out2)


def kernel(x, edge_index, batch, W_msg, b_msg, W_out, b_out):
    src = edge_index[0]
    dst = edge_index[1]
    zrows = jnp.zeros((RPT, D), jnp.float32)
    zdeg = jnp.zeros((N,), jnp.float32)
    agg_parts, deg_parts = _sc_scatter(x, src, dst, zrows, zdeg)
    deg_t = deg_parts.T
    batch2 = batch.reshape(N, 1)
    out = _dense(agg_parts, deg_t, batch2, W_msg,
                 b_msg.reshape(1, D), W_out, b_out.reshape(1, 1))
    return out.reshape(-1)


# speedup vs baseline: 11.3263x; 1.7539x over previous
import functools

import jax
import jax.numpy as jnp
from jax import lax
from jax.experimental import pallas as pl
from jax.experimental.pallas import tpu as pltpu
from jax.experimental.pallas import tpu_sc as plsc

N = 10000
E = 320000
D = 128
G = 64

NC = 2
NS = 16
NW = NC * NS
EPW = E // NW
CHUNK = 80
NCHUNK = EPW // CHUNK
NP = 10240
RPT = NP // NS


def _sc_scatter(x, src, dst, zrows, zdeg):
    mesh = plsc.VectorSubcoreMesh(
        core_axis_name="c", subcore_axis_name="s", num_cores=NC, num_subcores=NS
    )

    @functools.partial(
        pl.kernel,
        out_type=[
            jax.ShapeDtypeStruct((NC, NP, D), jnp.float32),
            jax.ShapeDtypeStruct((NW, N), jnp.float32),
        ],
        mesh=mesh,
        compiler_params=pltpu.CompilerParams(needs_layout_passes=False),
        scratch_types=[
            pltpu.VMEM((EPW,), jnp.int32),
            pltpu.VMEM((CHUNK,), jnp.int32),
            pltpu.VMEM((CHUNK,), jnp.int32),
            pltpu.VMEM((CHUNK, D), jnp.float32),
            pltpu.VMEM((CHUNK, D), jnp.float32),
            pltpu.VMEM((N,), jnp.float32),
            pltpu.VMEM_SHARED((NP, D), jnp.float32),
            pltpu.SemaphoreType.DMA,
            pltpu.SemaphoreType.DMA,
        ],
    )
    def k(x_hbm, src_hbm, dst_hbm, zr_hbm, zd_hbm, agg_out, deg_out,
          sidx, didx0, didx1, rows0, rows1, degh, aggsh, gsem, dsem):
        c = lax.axis_index("c")
        s = lax.axis_index("s")
        wid = s * NC + c
        base = wid * EPW
        rows = (rows0, rows1)
        didx = (didx0, didx1)

        pltpu.sync_copy(src_hbm.at[pl.ds(base, EPW)], sidx)
        pltpu.sync_copy(zr_hbm, aggsh.at[pl.ds(s * RPT, RPT)])
        pltpu.sync_copy(zd_hbm, degh)
        plsc.subcore_barrier()

        ones16 = jnp.ones((16,), jnp.float32)

        def start(j, b):
            off = pl.multiple_of(j * CHUNK, CHUNK)
            pltpu.async_copy(x_hbm.at[sidx.at[pl.ds(off, CHUNK)]],
                             rows[b], gsem)
            pltpu.async_copy(dst_hbm.at[pl.ds(base + off, CHUNK)],
                             didx[b], dsem)

        def process(j, b, prefetch):
            pltpu.make_async_copy(x_hbm.at[pl.ds(0, CHUNK)], rows[b],
                                  gsem).wait()
            pltpu.make_async_copy(dst_hbm.at[pl.ds(0, CHUNK)], didx[b],
                                  dsem).wait()
            if prefetch:
                start(j + 1, 1 - b)
            for t in range(CHUNK // 16):
                dv = didx[b][pl.ds(t * 16, 16)]
                plsc.addupdate_scatter(degh, [dv], ones16)
            pltpu.sync_copy(rows[b], aggsh.at[didx[b]], add=True)

        start(0, 0)

        def pair(j0, carry):
            process(j0, 0, True)
            process(j0 + 1, 1, True)
            return carry

        lax.fori_loop(0, (NCHUNK - 1) // 2, lambda i, cy: pair(2 * i, cy), 0)
        process(NCHUNK - 1, 0, False)

        plsc.subcore_barrier()

        pltpu.sync_copy(aggsh.at[pl.ds(s * RPT, RPT)],
                        agg_out.at[c, pl.ds(s * RPT, RPT)])
        pltpu.sync_copy(degh, deg_out.at[wid])

    return k(x, src, dst, zrows, zdeg)


_ROWS = 1000
_GRID = N // _ROWS


def _dense_body(agg_ref, deg_ref, b_ref, wm_ref, bm_ref, wo_ref, bo_ref,
                out_ref, accz, accc):
    i = pl.program_id(0)

    @pl.when(i == 0)
    def _init():
        accz[...] = jnp.zeros_like(accz)
        accc[...] = jnp.zeros_like(accc)

    agg = agg_ref[0] + agg_ref[1]
    deg = jnp.sum(deg_ref[...], axis=1, keepdims=True)
    agg = agg / jnp.maximum(deg, 1.0)
    h = jnp.dot(agg, wm_ref[...], preferred_element_type=jnp.float32)
    h = jnp.maximum(h + bm_ref[...], 0.0)
    z = jnp.dot(h, wo_ref[...], preferred_element_type=jnp.float32)
    onehot = (b_ref[...] == lax.broadcasted_iota(jnp.int32, (1, G), 1))
    onehot = onehot.astype(jnp.float32)
    accz[...] += jnp.sum(onehot * z, axis=0, keepdims=True)
    accc[...] += jnp.sum(onehot, axis=0, keepdims=True)

    @pl.when(i == _GRID - 1)
    def _fin():
        out_ref[...] = accz[...] / jnp.maximum(accc[...], 1.0) + bo_ref[...]


def _dense(agg_parts, deg_t, batch2, W_msg, b_msg2, W_out, b_out2):
    return pl.pallas_call(
        _dense_body,
        grid=(_GRID,),
        in_specs=[
            pl.BlockSpec((NC, _ROWS, D), lambda i: (0, i, 0)),
            pl.BlockSpec((_ROWS, NW), lambda i: (i, 0)),
            pl.BlockSpec((_ROWS, 1), lambda i: (i, 0)),
            pl.BlockSpec((D, D), lambda i: (0, 0)),
            pl.BlockSpec((1, D), lambda i: (0, 0)),
            pl.BlockSpec((D, 1), lambda i: (0, 0)),
            pl.BlockSpec((1, 1), lambda i: (0, 0)),
        ],
        out_specs=pl.BlockSpec((1, G), lambda i: (0, 0)),
        out_shape=jax.ShapeDtypeStruct((1, G), jnp.float32),
        scratch_shapes=[
            pltpu.VMEM((1, G), jnp.float32),
            pltpu.VMEM((1, G), jnp.float32),
        ],
    )(agg_parts, deg_t, batch2, W_msg, b_msg2, W_out, b_out2)


def kernel(x, edge_index, batch, W_msg, b_msg, W_out, b_out):
    src = edge_index[0]
    dst = edge_index[1]
    zrows = jnp.zeros((RPT, D), jnp.float32)
    zdeg = jnp.zeros((N,), jnp.float32)
    agg_parts, deg_parts = _sc_scatter(x, src, dst, zrows, zdeg)
    deg_t = deg_parts.T
    batch2 = batch.reshape(N, 1)
    out = _dense(agg_parts, deg_t, batch2, W_msg,
                 b_msg.reshape(1, D), W_out, b_out.reshape(1, 1))
    return out.reshape(-1)


# speedup vs baseline: 14.5432x; 1.2840x over previous
import functools

import jax
import jax.numpy as jnp
from jax import lax
from jax.experimental import pallas as pl
from jax.experimental.pallas import tpu as pltpu
from jax.experimental.pallas import tpu_sc as plsc

N = 10000
E = 320000
D = 128
G = 64

NC = 2
NS = 16
NW = NC * NS
EPW = E // NW
CHUNK = 80
NCHUNK = EPW // CHUNK
NP = 10240
RPT = NP // NS


def _sc_scatter(x, src, dst, zrows, zdeg):
    mesh = plsc.VectorSubcoreMesh(
        core_axis_name="c", subcore_axis_name="s", num_cores=NC, num_subcores=NS
    )

    @functools.partial(
        pl.kernel,
        out_type=[
            jax.ShapeDtypeStruct((NC, NP, D), jnp.float32),
            jax.ShapeDtypeStruct((NW, N), jnp.float32),
        ],
        mesh=mesh,
        compiler_params=pltpu.CompilerParams(needs_layout_passes=False),
        scratch_types=(
            [pltpu.VMEM((CHUNK,), jnp.int32)] * 6
            + [pltpu.VMEM((CHUNK,), jnp.int32)] * 6
            + [pltpu.VMEM((CHUNK, D), jnp.float32)] * 3
            + [
                pltpu.VMEM((N,), jnp.float32),
                pltpu.VMEM_SHARED((NP, D), jnp.float32),
                pltpu.SemaphoreType.DMA,
                pltpu.SemaphoreType.DMA,
                pltpu.SemaphoreType.DMA,
            ]
        ),
    )
    def k(x_hbm, src_hbm, dst_hbm, zr_hbm, zd_hbm, agg_out, deg_out,
          si0, si1, si2, si3, si4, si5, di0, di1, di2, di3, di4, di5,
          rw0, rw1, rw2, degh, aggsh, isem, gsem, ssem):
        c = lax.axis_index("c")
        s = lax.axis_index("s")
        wid = s * NC + c
        base = wid * EPW
        sidx = (si0, si1, si2, si3, si4, si5)
        didx = (di0, di1, di2, di3, di4, di5)
        rows = (rw0, rw1, rw2)

        pltpu.sync_copy(zr_hbm, aggsh.at[pl.ds(s * RPT, RPT)])
        pltpu.sync_copy(zd_hbm, degh)
        plsc.subcore_barrier()

        ones16 = jnp.ones((16,), jnp.float32)

        def idx_issue_u(j, b):
            off = base + j * CHUNK
            pltpu.async_copy(src_hbm.at[pl.ds(off, CHUNK)], sidx[b], isem)
            pltpu.async_copy(dst_hbm.at[pl.ds(off, CHUNK)], didx[b], isem)

        def idx_wait():
            pltpu.make_async_copy(src_hbm.at[pl.ds(0, CHUNK)], si0,
                                  isem).wait()
            pltpu.make_async_copy(src_hbm.at[pl.ds(0, CHUNK)], di0,
                                  isem).wait()

        def gather_issue(j, b6, b3):
            pltpu.async_copy(x_hbm.at[sidx[b6]], rows[b3], gsem)

        def gather_wait():
            pltpu.make_async_copy(x_hbm.at[pl.ds(0, CHUNK)], rw0, gsem).wait()

        def scatter_issue(b6, b3):
            pltpu.async_copy(rows[b3], aggsh.at[didx[b6]], ssem, add=True)

        def scatter_wait():
            pltpu.make_async_copy(rw0, aggsh.at[pl.ds(0, CHUNK)], ssem).wait()

        def deg_update(b6):
            for t in range(CHUNK // 16):
                dv = didx[b6][pl.ds(t * 16, 16)]
                plsc.addupdate_scatter(degh, [dv], ones16)

        def step(j, b6, b3, wait_s, issue_idx_j, wait_idx, issue_g_j):
            if wait_s:
                scatter_wait()
            if issue_idx_j is not None:
                idx_issue_u(issue_idx_j, (b6 + 3) % 6)
            if wait_idx:
                idx_wait()
            if issue_g_j is not None:
                gather_issue(issue_g_j, (b6 + 1) % 6, (b3 + 1) % 3)
            gather_wait()
            deg_update(b6)
            scatter_issue(b6, b3)

        idx_issue_u(0, 0)
        idx_issue_u(1, 1)
        idx_issue_u(2, 2)
        idx_wait()
        idx_wait()
        gather_issue(0, 0, 0)
        step(0, 0, 0, False, 3, True, 1)
        step(1, 1, 1, False, 4, True, 2)

        def six(i, carry):
            j0 = 2 + i * 6
            for u in range(6):
                b6 = (2 + u) % 6
                b3 = (2 + u) % 3
                step(j0 + u, b6, b3, True, j0 + u + 3, True, j0 + u + 1)
            return carry

        lax.fori_loop(0, 20, six, 0)

        step(122, (122 % 6), (122 % 3), True, None, True, 123)
        step(123, (123 % 6), (123 % 3), True, None, False, 124)
        step(124, (124 % 6), (124 % 3), True, None, False, None)
        scatter_wait()
        scatter_wait()

        plsc.subcore_barrier()

        pltpu.sync_copy(aggsh.at[pl.ds(s * RPT, RPT)],
                        agg_out.at[c, pl.ds(s * RPT, RPT)])
        pltpu.sync_copy(degh, deg_out.at[wid])

    return k(x, src, dst, zrows, zdeg)


_ROWS = 1000
_GRID = N // _ROWS


def _dense_body(agg_ref, deg_ref, b_ref, wm_ref, bm_ref, wo_ref, bo_ref,
                out_ref, accz, accc):
    i = pl.program_id(0)

    @pl.when(i == 0)
    def _init():
        accz[...] = jnp.zeros_like(accz)
        accc[...] = jnp.zeros_like(accc)

    agg = agg_ref[0] + agg_ref[1]
    deg = jnp.sum(deg_ref[...], axis=1, keepdims=True)
    agg = agg / jnp.maximum(deg, 1.0)
    h = jnp.dot(agg, wm_ref[...], preferred_element_type=jnp.float32)
    h = jnp.maximum(h + bm_ref[...], 0.0)
    z = jnp.dot(h, wo_ref[...], preferred_element_type=jnp.float32)
    onehot = (b_ref[...] == lax.broadcasted_iota(jnp.int32, (1, G), 1))
    onehot = onehot.astype(jnp.float32)
    accz[...] += jnp.sum(onehot * z, axis=0, keepdims=True)
    accc[...] += jnp.sum(onehot, axis=0, keepdims=True)

    @pl.when(i == _GRID - 1)
    def _fin():
        out_ref[...] = accz[...] / jnp.maximum(accc[...], 1.0) + bo_ref[...]


def _dense(agg_parts, deg_t, batch2, W_msg, b_msg2, W_out, b_out2):
    return pl.pallas_call(
        _dense_body,
        grid=(_GRID,),
        in_specs=[
            pl.BlockSpec((NC, _ROWS, D), lambda i: (0, i, 0)),
            pl.BlockSpec((_ROWS, NW), lambda i: (i, 0)),
            pl.BlockSpec((_ROWS, 1), lambda i: (i, 0)),
            pl.BlockSpec((D, D), lambda i: (0, 0)),
            pl.BlockSpec((1, D), lambda i: (0, 0)),
            pl.BlockSpec((D, 1), lambda i: (0, 0)),
            pl.BlockSpec((1, 1), lambda i: (0, 0)),
        ],
        out_specs=pl.BlockSpec((1, G), lambda i: (0, 0)),
        out_shape=jax.ShapeDtypeStruct((1, G), jnp.float32),
        scratch_shapes=[
            pltpu.VMEM((1, G), jnp.float32),
            pltpu.VMEM((1, G), jnp.float32),
        ],
    )(agg_parts, deg_t, batch2, W_msg, b_msg2, W_out, b_out2)


def kernel(x, edge_index, batch, W_msg, b_msg, W_out, b_out):
    src = edge_index[0]
    dst = edge_index[1]
    zrows = jnp.zeros((RPT, D), jnp.float32)
    zdeg = jnp.zeros((N,), jnp.float32)
    agg_parts, deg_parts = _sc_scatter(x, src, dst, zrows, zdeg)
    deg_t = deg_parts.T
    batch2 = batch.reshape(N, 1)
    out = _dense(agg_parts, deg_t, batch2, W_msg,
                 b_msg.reshape(1, D), W_out, b_out.reshape(1, 1))
    return out.reshape(-1)


# speedup vs baseline: 14.7050x; 1.0111x over previous
import functools

import jax
import jax.numpy as jnp
from jax import lax
from jax.experimental import pallas as pl
from jax.experimental.pallas import tpu as pltpu
from jax.experimental.pallas import tpu_sc as plsc

N = 10000
E = 320000
D = 128
G = 64

NC = 2
NS = 16
NW = NC * NS
EPW = E // NW
CHUNK = 80
NCHUNK = EPW // CHUNK
NP = 10240
RPT = NP // NS
DEGR = 80
DEG0 = 10080


def _sc_scatter(x, src, dst, zrows, rowids):
    mesh = plsc.VectorSubcoreMesh(
        core_axis_name="c", subcore_axis_name="s", num_cores=NC, num_subcores=NS
    )

    @functools.partial(
        pl.kernel,
        out_type=jax.ShapeDtypeStruct((NC, NP, D), jnp.float32),
        mesh=mesh,
        compiler_params=pltpu.CompilerParams(needs_layout_passes=False),
        scratch_types=(
            [pltpu.VMEM((CHUNK,), jnp.int32)] * 6
            + [pltpu.VMEM((CHUNK,), jnp.int32)] * 6
            + [pltpu.VMEM((CHUNK, D), jnp.float32)] * 3
            + [
                pltpu.VMEM((DEGR, D), jnp.float32),
                pltpu.VMEM((DEGR,), jnp.int32),
                pltpu.VMEM_SHARED((NP, D), jnp.float32),
                pltpu.SemaphoreType.DMA,
                pltpu.SemaphoreType.DMA,
                pltpu.SemaphoreType.DMA,
            ]
        ),
    )
    def k(x_hbm, src_hbm, dst_hbm, zr_hbm, ri_hbm, agg_out,
          si0, si1, si2, si3, si4, si5, di0, di1, di2, di3, di4, di5,
          rw0, rw1, rw2, degh, ri, aggsh, isem, gsem, ssem):
        c = lax.axis_index("c")
        s = lax.axis_index("s")
        wid = s * NC + c
        base = wid * EPW
        sidx = (si0, si1, si2, si3, si4, si5)
        didx = (di0, di1, di2, di3, di4, di5)
        rows = (rw0, rw1, rw2)

        pltpu.sync_copy(zr_hbm, aggsh.at[pl.ds(s * RPT, RPT)])
        pltpu.sync_copy(zr_hbm.at[pl.ds(0, DEGR)], degh)
        pltpu.sync_copy(ri_hbm, ri)
        plsc.subcore_barrier()

        ones16 = jnp.ones((16,), jnp.float32)

        def idx_issue_u(j, b):
            off = base + j * CHUNK
            pltpu.async_copy(src_hbm.at[pl.ds(off, CHUNK)], sidx[b], isem)
            pltpu.async_copy(dst_hbm.at[pl.ds(off, CHUNK)], didx[b], isem)

        def idx_wait():
            pltpu.make_async_copy(src_hbm.at[pl.ds(0, CHUNK)], si0,
                                  isem).wait()
            pltpu.make_async_copy(src_hbm.at[pl.ds(0, CHUNK)], di0,
                                  isem).wait()

        def gather_issue(j, b6, b3):
            pltpu.async_copy(x_hbm.at[sidx[b6]], rows[b3], gsem)

        def gather_wait():
            pltpu.make_async_copy(x_hbm.at[pl.ds(0, CHUNK)], rw0, gsem).wait()

        def scatter_issue(b6, b3):
            pltpu.async_copy(rows[b3], aggsh.at[didx[b6]], ssem, add=True)

        def scatter_wait():
            pltpu.make_async_copy(rw0, aggsh.at[pl.ds(0, CHUNK)], ssem).wait()

        def deg_update(b6):
            for t in range(CHUNK // 16):
                dv = didx[b6][pl.ds(t * 16, 16)]
                r = lax.shift_right_logical(dv, 7)
                col = lax.bitwise_and(dv, 127)
                plsc.addupdate_scatter(degh, [r, col], ones16)

        def step(j, b6, b3, wait_s, issue_idx_j, wait_idx, issue_g_j):
            if wait_s:
                scatter_wait()
            if issue_idx_j is not None:
                idx_issue_u(issue_idx_j, (b6 + 3) % 6)
            if wait_idx:
                idx_wait()
            if issue_g_j is not None:
                gather_issue(issue_g_j, (b6 + 1) % 6, (b3 + 1) % 3)
            gather_wait()
            deg_update(b6)
            scatter_issue(b6, b3)

        idx_issue_u(0, 0)
        idx_issue_u(1, 1)
        idx_issue_u(2, 2)
        idx_wait()
        idx_wait()
        gather_issue(0, 0, 0)
        step(0, 0, 0, False, 3, True, 1)
        step(1, 1, 1, False, 4, True, 2)

        def six(i, carry):
            j0 = 2 + i * 6
            for u in range(6):
                b6 = (2 + u) % 6
                b3 = (2 + u) % 3
                step(j0 + u, b6, b3, True, j0 + u + 3, True, j0 + u + 1)
            return carry

        lax.fori_loop(0, 20, six, 0)

        step(122, (122 % 6), (122 % 3), True, None, True, 123)
        step(123, (123 % 6), (123 % 3), True, None, False, 124)
        step(124, (124 % 6), (124 % 3), True, None, False, None)
        scatter_wait()
        scatter_wait()

        pltpu.sync_copy(degh, aggsh.at[ri], add=True)

        plsc.subcore_barrier()

        pltpu.sync_copy(aggsh.at[pl.ds(s * RPT, RPT)],
                        agg_out.at[c, pl.ds(s * RPT, RPT)])

    return k(x, src, dst, zrows, rowids)


_ROWS = 2000
_GRID = N // _ROWS


def _dense_body(agg_ref, deg_ref, b_ref, wm_ref, bm_ref, wo_ref, bo_ref,
                out_ref, accz, accc):
    i = pl.program_id(0)

    @pl.when(i == 0)
    def _init():
        accz[...] = jnp.zeros_like(accz)
        accc[...] = jnp.zeros_like(accc)

    agg = agg_ref[0] + agg_ref[1]
    deg = jnp.sum(deg_ref[...], axis=1, keepdims=True)
    agg = agg / jnp.maximum(deg, 1.0)
    h = jnp.dot(agg, wm_ref[...], preferred_element_type=jnp.float32)
    h = jnp.maximum(h + bm_ref[...], 0.0)
    z = jnp.dot(h, wo_ref[...], preferred_element_type=jnp.float32)
    onehot = (b_ref[...] == lax.broadcasted_iota(jnp.int32, (1, G), 1))
    onehot = onehot.astype(jnp.float32)
    accz[...] += jnp.sum(onehot * z, axis=0, keepdims=True)
    accc[...] += jnp.sum(onehot, axis=0, keepdims=True)

    @pl.when(i == _GRID - 1)
    def _fin():
        out_ref[...] = accz[...] / jnp.maximum(accc[...], 1.0) + bo_ref[...]


def _dense(agg_parts, deg_t, batch2, W_msg, b_msg2, W_out, b_out2):
    return pl.pallas_call(
        _dense_body,
        grid=(_GRID,),
        in_specs=[
            pl.BlockSpec((NC, _ROWS, D), lambda i: (0, i, 0)),
            pl.BlockSpec((_ROWS, NC), lambda i: (i, 0)),
            pl.BlockSpec((_ROWS, 1), lambda i: (i, 0)),
            pl.BlockSpec((D, D), lambda i: (0, 0)),
            pl.BlockSpec((1, D), lambda i: (0, 0)),
            pl.BlockSpec((D, 1), lambda i: (0, 0)),
            pl.BlockSpec((1, 1), lambda i: (0, 0)),
        ],
        out_specs=pl.BlockSpec((1, G), lambda i: (0, 0)),
        out_shape=jax.ShapeDtypeStruct((1, G), jnp.float32),
        scratch_shapes=[
            pltpu.VMEM((1, G), jnp.float32),
            pltpu.VMEM((1, G), jnp.float32),
        ],
    )(agg_parts, deg_t, batch2, W_msg, b_msg2, W_out, b_out2)


def kernel(x, edge_index, batch, W_msg, b_msg, W_out, b_out):
    src = edge_index[0]
    dst = edge_index[1]
    zrows = jnp.zeros((RPT, D), jnp.float32)
    rowids = jnp.arange(DEG0, DEG0 + DEGR, dtype=jnp.int32)
    agg_parts = _sc_scatter(x, src, dst, zrows, rowids)
    deg_t = agg_parts[:, DEG0:DEG0 + DEGR, :].reshape(NC, DEGR * D)[:, :N].T
    batch2 = batch.reshape(N, 1)
    out = _dense(agg_parts, deg_t, batch2, W_msg,
                 b_msg.reshape(1, D), W_out, b_out.reshape(1, 1))
    return out.reshape(-1)


# speedup vs baseline: 15.6464x; 1.0640x over previous
import functools

import jax
import jax.numpy as jnp
from jax import lax
from jax.experimental import pallas as pl
from jax.experimental.pallas import tpu as pltpu
from jax.experimental.pallas import tpu_sc as plsc

N = 10000
E = 320000
D = 128
G = 64

NC = 2
NS = 16
NW = NC * NS
EPW = E // NW
CHUNK = 80
NCHUNK = EPW // CHUNK
NP = 10240
RPT = NP // NS
DEGR = 80
DEG0 = 10080


def _sc_scatter(x, eidx, zrows, rowids):
    mesh = plsc.VectorSubcoreMesh(
        core_axis_name="c", subcore_axis_name="s", num_cores=NC, num_subcores=NS
    )

    @functools.partial(
        pl.kernel,
        out_type=jax.ShapeDtypeStruct((NC, NP, D), jnp.float32),
        mesh=mesh,
        compiler_params=pltpu.CompilerParams(needs_layout_passes=False),
        scratch_types=(
            [pltpu.VMEM((CHUNK,), jnp.int32)] * 6
            + [pltpu.VMEM((CHUNK,), jnp.int32)] * 6
            + [pltpu.VMEM((CHUNK, D), jnp.float32)] * 3
            + [
                pltpu.VMEM((DEGR, D), jnp.float32),
                pltpu.VMEM((DEGR,), jnp.int32),
                pltpu.VMEM_SHARED((NP, D), jnp.float32),
                pltpu.SemaphoreType.DMA,
                pltpu.SemaphoreType.DMA,
                pltpu.SemaphoreType.DMA,
            ]
        ),
    )
    def k(x_hbm, e_hbm, zr_hbm, ri_hbm, agg_out,
          si0, si1, si2, si3, si4, si5, di0, di1, di2, di3, di4, di5,
          rw0, rw1, rw2, degh, ri, aggsh, isem, gsem, ssem):
        c = lax.axis_index("c")
        s = lax.axis_index("s")
        wid = s * NC + c
        base = wid * EPW
        sidx = (si0, si1, si2, si3, si4, si5)
        didx = (di0, di1, di2, di3, di4, di5)
        rows = (rw0, rw1, rw2)

        pltpu.sync_copy(zr_hbm, aggsh.at[pl.ds(s * RPT, RPT)])
        pltpu.sync_copy(zr_hbm.at[pl.ds(0, DEGR)], degh)
        pltpu.sync_copy(ri_hbm, ri)
        plsc.subcore_barrier()

        ones16 = jnp.ones((16,), jnp.float32)

        def idx_issue_u(j, b):
            off = base + j * CHUNK
            pltpu.async_copy(e_hbm.at[pl.ds(off, CHUNK)], sidx[b], isem)
            pltpu.async_copy(e_hbm.at[pl.ds(E + off, CHUNK)], didx[b], isem)

        def idx_wait():
            pltpu.make_async_copy(e_hbm.at[pl.ds(0, CHUNK)], si0,
                                  isem).wait()
            pltpu.make_async_copy(e_hbm.at[pl.ds(0, CHUNK)], di0,
                                  isem).wait()

        def gather_issue(j, b6, b3):
            pltpu.async_copy(x_hbm.at[sidx[b6]], rows[b3], gsem)

        def gather_wait():
            pltpu.make_async_copy(x_hbm.at[pl.ds(0, CHUNK)], rw0, gsem).wait()

        def scatter_issue(b6, b3):
            pltpu.async_copy(rows[b3], aggsh.at[didx[b6]], ssem, add=True)

        def scatter_wait():
            pltpu.make_async_copy(rw0, aggsh.at[pl.ds(0, CHUNK)], ssem).wait()

        def deg_update(b6):
            for t in range(CHUNK // 16):
                dv = didx[b6][pl.ds(t * 16, 16)]
                r = lax.shift_right_logical(dv, 7)
                col = lax.bitwise_and(dv, 127)
                plsc.addupdate_scatter(degh, [r, col], ones16)

        def step(j, b6, b3, wait_s, issue_idx_j, wait_idx, issue_g_j):
            if wait_s:
                scatter_wait()
            if issue_idx_j is not None:
                idx_issue_u(issue_idx_j, (b6 + 3) % 6)
            if wait_idx:
                idx_wait()
            if issue_g_j is not None:
                gather_issue(issue_g_j, (b6 + 1) % 6, (b3 + 1) % 3)
            gather_wait()
            deg_update(b6)
            scatter_issue(b6, b3)

        idx_issue_u(0, 0)
        idx_issue_u(1, 1)
        idx_issue_u(2, 2)
        idx_wait()
        idx_wait()
        gather_issue(0, 0, 0)
        step(0, 0, 0, False, 3, True, 1)
        step(1, 1, 1, False, 4, True, 2)

        def six(i, carry):
            j0 = 2 + i * 6
            for u in range(6):
                b6 = (2 + u) % 6
                b3 = (2 + u) % 3
                step(j0 + u, b6, b3, True, j0 + u + 3, True, j0 + u + 1)
            return carry

        lax.fori_loop(0, 20, six, 0)

        step(122, (122 % 6), (122 % 3), True, None, True, 123)
        step(123, (123 % 6), (123 % 3), True, None, False, 124)
        step(124, (124 % 6), (124 % 3), True, None, False, None)
        scatter_wait()
        scatter_wait()

        pltpu.sync_copy(degh, aggsh.at[ri], add=True)

        plsc.subcore_barrier()

        pltpu.sync_copy(aggsh.at[pl.ds(s * RPT, RPT)],
                        agg_out.at[c, pl.ds(s * RPT, RPT)])

    return k(x, eidx, zrows, rowids)


_ROWS = 2000
_GRID = N // _ROWS


def _dense_body(agg_ref, deg_ref, b_ref, wm_ref, bm_ref, wo_ref, bo_ref,
                out_ref, accz, accc):
    i = pl.program_id(0)

    @pl.when(i == 0)
    def _init():
        accz[...] = jnp.zeros_like(accz)
        accc[...] = jnp.zeros_like(accc)

    agg = agg_ref[0] + agg_ref[1]
    deg = jnp.sum(deg_ref[...], axis=1, keepdims=True)
    agg = agg / jnp.maximum(deg, 1.0)
    h = jnp.dot(agg, wm_ref[...], preferred_element_type=jnp.float32)
    h = jnp.maximum(h + bm_ref[...], 0.0)
    z = jnp.dot(h, wo_ref[...], preferred_element_type=jnp.float32)
    onehot = (b_ref[...] == lax.broadcasted_iota(jnp.int32, (1, G), 1))
    onehot = onehot.astype(jnp.float32)
    accz[...] += jnp.sum(onehot * z, axis=0, keepdims=True)
    accc[...] += jnp.sum(onehot, axis=0, keepdims=True)

    @pl.when(i == _GRID - 1)
    def _fin():
        out_ref[...] = accz[...] / jnp.maximum(accc[...], 1.0) + bo_ref[...]


def _dense(agg_parts, deg_t, batch2, W_msg, b_msg2, W_out, b_out2):
    return pl.pallas_call(
        _dense_body,
        grid=(_GRID,),
        in_specs=[
            pl.BlockSpec((NC, _ROWS, D), lambda i: (0, i, 0)),
            pl.BlockSpec((_ROWS, NC), lambda i: (i, 0)),
            pl.BlockSpec((_ROWS, 1), lambda i: (i, 0)),
            pl.BlockSpec((D, D), lambda i: (0, 0)),
            pl.BlockSpec((1, D), lambda i: (0, 0)),
            pl.BlockSpec((D, 1), lambda i: (0, 0)),
            pl.BlockSpec((1, 1), lambda i: (0, 0)),
        ],
        out_specs=pl.BlockSpec((1, G), lambda i: (0, 0)),
        out_shape=jax.ShapeDtypeStruct((1, G), jnp.float32),
        scratch_shapes=[
            pltpu.VMEM((1, G), jnp.float32),
            pltpu.VMEM((1, G), jnp.float32),
        ],
    )(agg_parts, deg_t, batch2, W_msg, b_msg2, W_out, b_out2)


def kernel(x, edge_index, batch, W_msg, b_msg, W_out, b_out):
    eidx = edge_index.reshape(2 * E)
    zrows = jnp.zeros((RPT, D), jnp.float32)
    rowids = jnp.arange(DEG0, DEG0 + DEGR, dtype=jnp.int32)
    agg_parts = _sc_scatter(x, eidx, zrows, rowids)
    deg_t = agg_parts[:, DEG0:DEG0 + DEGR, :].reshape(NC, DEGR * D)[:, :N].T
    batch2 = batch.reshape(N, 1)
    out = _dense(agg_parts, deg_t, batch2, W_msg,
                 b_msg.reshape(1, D), W_out, b_out.reshape(1, 1))
    return out.reshape(-1)
